# baseline (speedup 1.0000x reference)
import jax
import jax.numpy as jnp
from jax import lax
from jax.experimental import pallas as pl
from jax.experimental.pallas import tpu as pltpu
from jax.experimental.pallas import tpu_sc as plsc

N_NODES = 10000
N_EDGES = 320000
D = 128
HALF = N_NODES // 2

NC = 2
NS = 16

CHUNK = 32
N_ROWS = 10240
E_PAD = N_ROWS * CHUNK
ROWS_PER_TILE = N_ROWS // NS
X_ROWS = 10008
ZERO_SRC = 10000
ACC_ROWS = 5120
DUMMY_DST = N_NODES


def _matmul_body(h_ref, w_ref, norm_ref, o_ref):
    o_ref[...] = (
        jnp.dot(h_ref[...], w_ref[...], preferred_element_type=jnp.float32)
        * norm_ref[...]
    )


def _matmul(h, W, norm):
    blk = 2000
    grid = (N_NODES // blk,)
    return pl.pallas_call(
        _matmul_body,
        grid=grid,
        in_specs=[
            pl.BlockSpec((blk, D), lambda i: (i, 0)),
            pl.BlockSpec((D, D), lambda i: (0, 0)),
            pl.BlockSpec((blk, 1), lambda i: (i, 0)),
        ],
        out_specs=pl.BlockSpec((blk, D), lambda i: (i, 0)),
        out_shape=jax.ShapeDtypeStruct((N_NODES, D), jnp.float32),
    )(h, W, norm)


def _sc_body(idx_hbm, x_hbm, zeros_hbm, out_hbm,
             raw_idx, src_idx, dst_idx, rows_a, rows_b, xsp, acc, *sems):
    cid = lax.axis_index("c")
    sid = lax.axis_index("s")
    bufs = (rows_a, rows_b)
    rsem = sems[0:2]
    isem = sems[2:6]
    last = ROWS_PER_TILE - 1

    @pl.when(sid < 15)
    def _():
        pltpu.sync_copy(x_hbm.at[pl.ds(sid * 632, 632)],
                        xsp.at[pl.ds(sid * 632, 632)])

    @pl.when(sid == 15)
    def _():
        pltpu.sync_copy(x_hbm.at[pl.ds(9480, 520)], xsp.at[pl.ds(9480, 520)])
        pltpu.sync_copy(zeros_hbm.at[pl.ds(0, 8)], xsp.at[pl.ds(10000, 8)])

    pltpu.sync_copy(zeros_hbm, acc.at[pl.ds(sid * 320, 128)])
    pltpu.sync_copy(zeros_hbm, acc.at[pl.ds(sid * 320 + 128, 128)])
    pltpu.sync_copy(zeros_hbm.at[pl.ds(0, 64)],
                    acc.at[pl.ds(sid * 320 + 256, 64)])
    plsc.subcore_barrier()

    def idx_start(t, slot):
        base = sid * ROWS_PER_TILE
        pltpu.async_copy(idx_hbm.at[base + t], raw_idx.at[pl.ds(slot, 1)],
                         isem[slot])

    def idx_wait(t, slot):
        base = sid * ROWS_PER_TILE
        pltpu.make_async_copy(idx_hbm.at[base + t],
                              raw_idx.at[pl.ds(slot, 1)], isem[slot]).wait()

    def transform(slot):
        lo = cid * HALF
        for v in range(CHUNK // 16):
            sl = raw_idx[slot, pl.ds(v * 16, 16)]
            dl = raw_idx[slot, pl.ds(CHUNK + v * 16, 16)]
            local = dl - lo
            invalid = (local < 0) | (local >= HALF)
            dst_idx[slot, pl.ds(v * 16, 16)] = jnp.where(
                invalid, dl & 4095, local)
            src_idx[slot, pl.ds(v * 16, 16)] = jnp.where(
                invalid, ZERO_SRC, sl)

    def gather_start(t, islot, slot):
        pltpu.async_copy(xsp.at[src_idx.at[islot]], bufs[slot], rsem[slot])

    def gather_wait(t, islot, slot):
        pltpu.make_async_copy(xsp.at[src_idx.at[islot]], bufs[slot],
                              rsem[slot]).wait()

    for i in range(4):
        idx_start(i, i)
    for i in range(2):
        idx_wait(i, i)
        transform(i)
        gather_start(i, i, i)

    def body(g, carry):
        for i in range(4):
            t = g * 4 + i
            gather_wait(t, i, i % 2)
            pltpu.sync_copy(bufs[i % 2], acc.at[dst_idx.at[i]], add=True)
            idx_wait(jnp.minimum(t + 2, last), (i + 2) % 4)
            transform((i + 2) % 4)
            gather_start(jnp.minimum(t + 2, last), (i + 2) % 4, i % 2)
            idx_start(jnp.minimum(t + 4, last), i)
        return carry

    lax.fori_loop(0, ROWS_PER_TILE // 4, body, 0, unroll=False)
    for i in range(2):
        gather_wait(last, (i + 2) % 4, i)
    for i in (2, 3):
        idx_wait(last, i)
    plsc.subcore_barrier()

    pltpu.sync_copy(acc.at[pl.ds(sid * 320, 320)],
                    out_hbm.at[cid, pl.ds(sid * 320, 320)])


def _sc_scatter(idx3d, x, zeros):
    mesh = plsc.VectorSubcoreMesh(core_axis_name="c", subcore_axis_name="s")
    f = pl.kernel(
        _sc_body,
        out_type=jax.ShapeDtypeStruct((NC, ACC_ROWS, D), jnp.float32),
        mesh=mesh,
        scratch_types=[
            pltpu.VMEM((4, 2 * CHUNK), jnp.int32),
            pltpu.VMEM((4, CHUNK), jnp.int32),
            pltpu.VMEM((4, CHUNK), jnp.int32),
            pltpu.VMEM((CHUNK, D), jnp.float32),
            pltpu.VMEM((CHUNK, D), jnp.float32),
            pltpu.VMEM_SHARED((X_ROWS, D), jnp.float32),
            pltpu.VMEM_SHARED((ACC_ROWS, D), jnp.float32),
        ] + [pltpu.SemaphoreType.DMA] * 6,
    )
    return f(idx3d, x, zeros)


def _finish_body(p_ref, norm_ref, b_ref, o_ref):
    o_ref[...] = p_ref[0] * norm_ref[...] + b_ref[...]


def _finish(partials, norm, b):
    blk = 1000
    grid = (N_NODES // blk,)
    return pl.pallas_call(
        _finish_body,
        grid=grid,
        in_specs=[
            pl.BlockSpec((1, blk, D), lambda i: (i // 5, i % 5, 0)),
            pl.BlockSpec((blk, 1), lambda i: (i, 0)),
            pl.BlockSpec((1, D), lambda i: (0, 0)),
        ],
        out_specs=pl.BlockSpec((blk, D), lambda i: (i, 0)),
        out_shape=jax.ShapeDtypeStruct((N_NODES, D), jnp.float32),
    )(partials, norm, b.reshape(1, D))


def kernel(h, edge_index, norm, W, b):
    ei = edge_index.astype(jnp.int32)
    pad = E_PAD - N_EDGES
    src = jnp.concatenate([ei[0], jnp.zeros((pad,), jnp.int32)])
    dst = jnp.concatenate([ei[1], jnp.full((pad,), DUMMY_DST, jnp.int32)])
    idx3d = jnp.concatenate(
        [src.reshape(N_ROWS, 1, CHUNK), dst.reshape(N_ROWS, 1, CHUNK)],
        axis=2)
    zeros = jnp.zeros((128, D), jnp.float32)

    x = _matmul(h, W, norm)
    partials = _sc_scatter(idx3d, x, zeros)
    out = _finish(partials, norm, b)
    return out


# speedup vs baseline: 1.0359x; 1.0359x over previous
import jax
import jax.numpy as jnp
from jax import lax
from jax.experimental import pallas as pl
from jax.experimental.pallas import tpu as pltpu
from jax.experimental.pallas import tpu_sc as plsc

N_NODES = 10000
N_EDGES = 320000
D = 128
HALF = N_NODES // 2

NC = 2
NS = 16

CHUNK = 32
N_ROWS = 10240
E_PAD = N_ROWS * CHUNK
ROWS_PER_TILE = N_ROWS // NS
X_ROWS = 10112
ZERO_SRC = 10104
ACC_ROWS = 5120
DUMMY_DST = N_NODES


def _matmul_body(h_ref, w_ref, norm_ref, o_ref):
    o_ref[...] = (
        jnp.dot(h_ref[...], w_ref[...], preferred_element_type=jnp.float32)
        * norm_ref[...]
    )


def _matmul(h, W, norm):
    blk = 2000
    grid = (N_NODES // blk,)
    return pl.pallas_call(
        _matmul_body,
        grid=grid,
        in_specs=[
            pl.BlockSpec((blk, D), lambda i: (i, 0)),
            pl.BlockSpec((D, D), lambda i: (0, 0)),
            pl.BlockSpec((blk, 1), lambda i: (i, 0)),
        ],
        out_specs=pl.BlockSpec((blk, D), lambda i: (i, 0)),
        out_shape=jax.ShapeDtypeStruct((N_NODES, D), jnp.float32),
    )(h, W, norm)


def _sc_body(x_hbm, src_hbm, dst_hbm, zeros_hbm, out_hbm,
             src_idx, dst_idx, rows_a, rows_b, xsp, acc, *sems):
    cid = lax.axis_index("c")
    sid = lax.axis_index("s")
    bufs = (rows_a, rows_b)
    rsem = sems[0:2]
    ssem = sems[2:6]
    dsem = sems[6:10]
    last = ROWS_PER_TILE - 1

    @pl.when(sid < 15)
    def _():
        pltpu.sync_copy(x_hbm.at[pl.ds(sid * 632, 632)],
                        xsp.at[pl.ds(sid * 632, 632)])

    @pl.when(sid == 15)
    def _():
        pltpu.sync_copy(x_hbm.at[pl.ds(9480, 520)], xsp.at[pl.ds(9480, 520)])
        pltpu.sync_copy(zeros_hbm.at[pl.ds(0, 112)],
                        xsp.at[pl.ds(10000, 112)])

    pltpu.sync_copy(zeros_hbm, acc.at[pl.ds(sid * 320, 128)])
    pltpu.sync_copy(zeros_hbm, acc.at[pl.ds(sid * 320 + 128, 128)])
    pltpu.sync_copy(zeros_hbm.at[pl.ds(0, 64)],
                    acc.at[pl.ds(sid * 320 + 256, 64)])
    plsc.subcore_barrier()

    def idx_start(t, slot):
        base = sid * ROWS_PER_TILE
        pltpu.async_copy(src_hbm.at[base + t], src_idx.at[pl.ds(slot, 1)],
                         ssem[slot])
        pltpu.async_copy(dst_hbm.at[base + t], dst_idx.at[pl.ds(slot, 1)],
                         dsem[slot])

    def idx_wait(t, slot):
        base = sid * ROWS_PER_TILE
        pltpu.make_async_copy(src_hbm.at[base + t],
                              src_idx.at[pl.ds(slot, 1)], ssem[slot]).wait()
        pltpu.make_async_copy(dst_hbm.at[base + t],
                              dst_idx.at[pl.ds(slot, 1)], dsem[slot]).wait()

    def transform(slot):
        lo = cid * HALF
        for v in range(CHUNK // 16):
            sl = src_idx[slot, pl.ds(v * 16, 16)]
            dl = dst_idx[slot, pl.ds(v * 16, 16)]
            local = dl - lo
            invalid = (local < 0) | (local >= HALF)
            dst_idx[slot, pl.ds(v * 16, 16)] = jnp.where(
                invalid, dl & 4095, local)
            src_idx[slot, pl.ds(v * 16, 16)] = jnp.where(
                invalid, ZERO_SRC, sl)

    def gather_start(t, islot, slot):
        pltpu.async_copy(xsp.at[src_idx.at[islot]], bufs[slot], rsem[slot])

    def gather_wait(t, islot, slot):
        pltpu.make_async_copy(xsp.at[src_idx.at[islot]], bufs[slot],
                              rsem[slot]).wait()

    for i in range(4):
        idx_start(i, i)
    for i in range(2):
        idx_wait(i, i)
        transform(i)
        gather_start(i, i, i)

    def body(g, carry):
        for i in range(4):
            t = g * 4 + i
            gather_wait(t, i, i % 2)
            pltpu.sync_copy(bufs[i % 2], acc.at[dst_idx.at[i]], add=True)
            idx_wait(jnp.minimum(t + 2, last), (i + 2) % 4)
            transform((i + 2) % 4)
            gather_start(jnp.minimum(t + 2, last), (i + 2) % 4, i % 2)
            idx_start(jnp.minimum(t + 4, last), i)
        return carry

    lax.fori_loop(0, ROWS_PER_TILE // 4, body, 0, unroll=False)
    for i in range(2):
        gather_wait(last, (i + 2) % 4, i)
    for i in (2, 3):
        idx_wait(last, i)
    plsc.subcore_barrier()

    pltpu.sync_copy(acc.at[pl.ds(sid * 320, 320)],
                    out_hbm.at[cid, pl.ds(sid * 320, 320)])


def _sc_scatter(x, src3d, dst3d, zeros):
    mesh = plsc.VectorSubcoreMesh(core_axis_name="c", subcore_axis_name="s")
    f = pl.kernel(
        _sc_body,
        out_type=jax.ShapeDtypeStruct((NC, ACC_ROWS, D), jnp.float32),
        mesh=mesh,
        scratch_types=[
            pltpu.VMEM((4, CHUNK), jnp.int32),
            pltpu.VMEM((4, CHUNK), jnp.int32),
            pltpu.VMEM((CHUNK, D), jnp.float32),
            pltpu.VMEM((CHUNK, D), jnp.float32),
            pltpu.VMEM_SHARED((X_ROWS, D), jnp.float32),
            pltpu.VMEM_SHARED((ACC_ROWS, D), jnp.float32),
        ] + [pltpu.SemaphoreType.DMA] * 10,
    )
    return f(x, src3d, dst3d, zeros)


def _finish_body(p_ref, norm_ref, b_ref, o_ref):
    o_ref[...] = p_ref[0] * norm_ref[...] + b_ref[...]


def _finish(partials, norm, b):
    blk = 1000
    grid = (N_NODES // blk,)
    return pl.pallas_call(
        _finish_body,
        grid=grid,
        in_specs=[
            pl.BlockSpec((1, blk, D), lambda i: (i // 5, i % 5, 0)),
            pl.BlockSpec((blk, 1), lambda i: (i, 0)),
            pl.BlockSpec((1, D), lambda i: (0, 0)),
        ],
        out_specs=pl.BlockSpec((blk, D), lambda i: (i, 0)),
        out_shape=jax.ShapeDtypeStruct((N_NODES, D), jnp.float32),
    )(partials, norm, b.reshape(1, D))


def kernel(h, edge_index, norm, W, b):
    ei = edge_index.astype(jnp.int32)
    pad = E_PAD - N_EDGES
    src = jnp.concatenate([ei[0], jnp.zeros((pad,), jnp.int32)])
    dst = jnp.concatenate([ei[1], jnp.full((pad,), DUMMY_DST, jnp.int32)])
    src3d = src.reshape(N_ROWS, 1, CHUNK)
    dst3d = dst.reshape(N_ROWS, 1, CHUNK)
    zeros = jnp.zeros((128, D), jnp.float32)

    x = _matmul(h, W, norm)
    partials = _sc_scatter(x, src3d, dst3d, zeros)
    out = _finish(partials, norm, b)
    return out
